# Initial kernel scaffold; baseline (speedup 1.0000x reference)
import jax
import jax.numpy as jnp
from jax.experimental import pallas as pl


def kernel(x, edge_index, W1, b1, W2, b2, Wm1, bm1, Wm2, bm2):
    raise NotImplementedError("write your pallas kernel here")


# speedup vs baseline: 10.3734x; 10.3734x over previous
import functools

import jax
import jax.numpy as jnp
from jax import lax
from jax.experimental import pallas as pl
from jax.experimental.pallas import tpu as pltpu
from jax.experimental.pallas import tpu_sc as plsc

NC = 2
NS = 16
NW = NC * NS
L = 16

G = 125
GE = 80

_mesh = functools.partial(
    plsc.VectorSubcoreMesh,
    core_axis_name="c", subcore_axis_name="s", num_cores=NC, num_subcores=NS,
)


def _make_sc_degree(n_pad, e):
    gpt = e // NS // G
    rpt = n_pad // NS

    @functools.partial(
        pl.kernel,
        out_type=jax.ShapeDtypeStruct((n_pad,), jnp.float32),
        mesh=_mesh(),
        scratch_types=[
            pltpu.VMEM((gpt, G), jnp.int32),
            pltpu.VMEM((G,), jnp.float32),
            pltpu.VMEM((rpt,), jnp.float32),
            pltpu.VMEM_SHARED((n_pad,), jnp.float32),
        ],
    )
    def sc_degree(dst_hbm, ones_hbm, zeros_hbm, cnt_hbm, idx_v, ones_v, buf_v,
                  cnt_sh):
        cid = lax.axis_index("c")
        sid = lax.axis_index("s")

        @pl.when(cid == 0)
        def _():
            base = sid * rpt
            pltpu.sync_copy(zeros_hbm.at[pl.ds(0, rpt)], buf_v)
            pltpu.sync_copy(buf_v, cnt_sh.at[pl.ds(base, rpt)])
            pltpu.sync_copy(ones_hbm, ones_v)
            pltpu.sync_copy(dst_hbm.at[sid], idx_v)
            plsc.subcore_barrier()

            def body(g, _):
                pltpu.sync_copy(ones_v, cnt_sh.at[idx_v.at[g]], add=True)
                return 0

            lax.fori_loop(0, gpt, body, 0)
            plsc.subcore_barrier()
            pltpu.sync_copy(cnt_sh.at[pl.ds(base, rpt)],
                            cnt_hbm.at[pl.ds(base, rpt)])

    return sc_degree


def _make_sc_scatter(n_pad, e, d):
    gpt = e // NW // G
    rpt = n_pad // NS

    @functools.partial(
        pl.kernel,
        out_type=jax.ShapeDtypeStruct((NC, n_pad, d), jnp.float32),
        mesh=_mesh(),
        scratch_types=[
            pltpu.VMEM((gpt, G), jnp.int32),
            pltpu.VMEM((gpt, G), jnp.int32),
            pltpu.VMEM((G, d), jnp.float32),
            pltpu.VMEM_SHARED((n_pad, d), jnp.float32),
            pltpu.SemaphoreType.DMA,
        ],
    )
    def sc_scatter(y_hbm, src_hbm, dst_hbm, zrows_hbm, agg_hbm, sidx, didx,
                   rows, tab, sem):
        cid = lax.axis_index("c")
        sid = lax.axis_index("s")
        w = cid * NS + sid
        base = sid * rpt

        pltpu.sync_copy(zrows_hbm, tab.at[pl.ds(base, rpt)])
        pltpu.sync_copy(src_hbm.at[w], sidx)
        pltpu.sync_copy(dst_hbm.at[w], didx)
        plsc.subcore_barrier()

        def body(g, _):
            pltpu.async_copy(y_hbm.at[sidx.at[g]], rows, sem).wait()
            pltpu.sync_copy(rows, tab.at[didx.at[g]], add=True)
            return 0

        lax.fori_loop(0, gpt, body, 0)
        plsc.subcore_barrier()
        pltpu.sync_copy(tab.at[pl.ds(base, rpt)],
                        agg_hbm.at[cid, pl.ds(base, rpt)])

    return sc_scatter


def _make_sc_edge(n_pad, e, d):
    gpt = e // NW // GE
    ept = e // NW

    @functools.partial(
        pl.kernel,
        out_type=jax.ShapeDtypeStruct((e, d), jnp.float32),
        mesh=_mesh(),
        scratch_types=[
            pltpu.VMEM((gpt, GE), jnp.int32),
            pltpu.VMEM((gpt, GE), jnp.int32),
            pltpu.VMEM((GE, d), jnp.float32),
            pltpu.VMEM((GE, d), jnp.float32),
            pltpu.SemaphoreType.DMA,
            pltpu.SemaphoreType.DMA,
        ],
    )
    def sc_edge(a_hbm, b_hbm, src_hbm, dst_hbm, t_hbm, sidx, didx, arows,
                brows, sema, semb):
        cid = lax.axis_index("c")
        sid = lax.axis_index("s")
        w = cid * NS + sid

        pltpu.sync_copy(src_hbm.at[w], sidx)
        pltpu.sync_copy(dst_hbm.at[w], didx)

        def body(g, _):
            ca = pltpu.async_copy(a_hbm.at[sidx.at[g]], arows, sema)
            cb = pltpu.async_copy(b_hbm.at[didx.at[g]], brows, semb)
            ca.wait()
            cb.wait()

            def row(r, _):
                for j in range(d // L):
                    s = pl.ds(j * L, L)
                    arows[r, s] = jnp.maximum(arows[r, s] + brows[r, s], 0.0)
                return 0

            lax.fori_loop(0, GE, row, 0)
            pltpu.sync_copy(arows, t_hbm.at[pl.ds(w * ept + g * GE, GE)])
            return 0

        lax.fori_loop(0, gpt, body, 0)

    return sc_edge


def _tc1_body(x_ref, w_ref, cnt_ref, y_ref):
    dis = lax.rsqrt(1.0 + cnt_ref[...])
    xw = jnp.dot(x_ref[...], w_ref[...], preferred_element_type=jnp.float32)
    y_ref[...] = dis * xw


def _tc2_body(a0_ref, a1_ref, y_ref, cnt_ref, b_ref, w_ref, o_ref):
    dis = lax.rsqrt(1.0 + cnt_ref[...])
    h = jnp.maximum(dis * (a0_ref[...] + a1_ref[...] + y_ref[...])
                    + b_ref[...], 0.0)
    o_ref[...] = dis * jnp.dot(h, w_ref[...],
                               preferred_element_type=jnp.float32)


def _tc3_body(a0_ref, a1_ref, y_ref, cnt_ref, b_ref, wt_ref, wb_ref, bm_ref,
              ao_ref, bo_ref):
    dis = lax.rsqrt(1.0 + cnt_ref[...])
    h = jnp.maximum(dis * (a0_ref[...] + a1_ref[...] + y_ref[...])
                    + b_ref[...], 0.0)
    ao_ref[...] = jnp.dot(h, wt_ref[...],
                          preferred_element_type=jnp.float32) + bm_ref[...]
    bo_ref[...] = jnp.dot(h, wb_ref[...], preferred_element_type=jnp.float32)


def _tc4_body(t_ref, w_ref, b_ref, z_ref):
    z_ref[...] = jnp.dot(t_ref[...], w_ref[...],
                         preferred_element_type=jnp.float32) + b_ref[...]


def _row_spec(r, c):
    return pl.BlockSpec((r, c), lambda i: (i, 0))


def _full_spec(r, c):
    return pl.BlockSpec((r, c), lambda i: (0, 0))


def kernel(x, edge_index, W1, b1, W2, b2, Wm1, bm1, Wm2, bm2):
    n, d_in = x.shape
    d_hid = W1.shape[1]
    d_lat = Wm2.shape[1]
    e = edge_index.shape[1]

    n_pad = ((n + NW * L - 1) // (NW * L)) * (NW * L)

    x_pad = jnp.concatenate(
        [x, jnp.zeros((n_pad - n, d_in), jnp.float32)], axis=0)
    src = edge_index[0].reshape(NW, e // NW // G, G)
    dst = edge_index[1].reshape(NW, e // NW // G, G)
    dst16 = edge_index[1].reshape(NS, e // NS // G, G)

    ones_g = jnp.ones((G,), jnp.float32)
    zeros_r = jnp.zeros((n_pad // NS,), jnp.float32)
    zrows = jnp.zeros((n_pad // NS, d_hid), jnp.float32)

    cnt1 = _make_sc_degree(n_pad, e)(dst16, ones_g, zeros_r)
    cnt = cnt1.reshape(n_pad, 1)

    R = 512
    grid_n = (n_pad // R,)
    dis_spec = pl.BlockSpec((R, 1), lambda i: (i, 0))

    y1 = pl.pallas_call(
        _tc1_body,
        grid=grid_n,
        in_specs=[_row_spec(R, d_in), _full_spec(d_in, d_hid), dis_spec],
        out_specs=_row_spec(R, d_hid),
        out_shape=jax.ShapeDtypeStruct((n_pad, d_hid), jnp.float32),
    )(x_pad, W1, cnt)

    sc_scatter = _make_sc_scatter(n_pad, e, d_hid)
    agg1 = sc_scatter(y1, src, dst, zrows)

    tc_mid_specs = [
        _row_spec(R, d_hid), _row_spec(R, d_hid), _row_spec(R, d_hid),
        dis_spec, _full_spec(1, d_hid),
    ]
    y2 = pl.pallas_call(
        _tc2_body,
        grid=grid_n,
        in_specs=tc_mid_specs + [_full_spec(d_hid, d_hid)],
        out_specs=_row_spec(R, d_hid),
        out_shape=jax.ShapeDtypeStruct((n_pad, d_hid), jnp.float32),
    )(agg1[0], agg1[1], y1, cnt, b1.reshape(1, d_hid), W2)

    agg2 = sc_scatter(y2, src, dst, zrows)

    A, B = pl.pallas_call(
        _tc3_body,
        grid=grid_n,
        in_specs=tc_mid_specs + [
            _full_spec(d_hid, d_hid), _full_spec(d_hid, d_hid),
            _full_spec(1, d_hid),
        ],
        out_specs=[_row_spec(R, d_hid), _row_spec(R, d_hid)],
        out_shape=[jax.ShapeDtypeStruct((n_pad, d_hid), jnp.float32),
                   jax.ShapeDtypeStruct((n_pad, d_hid), jnp.float32)],
    )(agg2[0], agg2[1], y2, cnt, b2.reshape(1, d_hid),
      Wm1[:d_hid], Wm1[d_hid:], bm1.reshape(1, d_hid))

    src_e = edge_index[0].reshape(NW, e // NW // GE, GE)
    dst_e = edge_index[1].reshape(NW, e // NW // GE, GE)
    t = _make_sc_edge(n_pad, e, d_hid)(A, B, src_e, dst_e)

    RE = 2000
    z = pl.pallas_call(
        _tc4_body,
        grid=(e // RE,),
        in_specs=[_row_spec(RE, d_hid), _full_spec(d_hid, d_lat),
                  _full_spec(1, d_lat)],
        out_specs=_row_spec(RE, d_lat),
        out_shape=jax.ShapeDtypeStruct((e, d_lat), jnp.float32),
    )(t, Wm2, bm2.reshape(1, d_lat))

    return z
